# baseline (speedup 1.0000x reference)
import jax
import jax.numpy as jnp
from jax.experimental import pallas as pl

F32 = jnp.float32
R = 930
ROWS = 840
OFF = 31


def _blk_conv(x, w_stack, b_row, block_taps, act, cout, pad_out=True,
              out_dtype=F32):
    B, nbi, _, cin = x.shape
    w_stack = w_stack.astype(x.dtype)
    nbo = len(block_taps)
    T = w_stack.shape[0]
    rout = R if pad_out else ROWS

    def kfn(x_ref, w_ref, b_ref, o_ref):
        kk = jax.lax.broadcasted_iota(jnp.int32, (ROWS, 1), 0)
        msk = (kk % 30) < 28
        for bo, taps in enumerate(block_taps):
            acc = jnp.zeros((ROWS, cout), F32)
            for (bi, off, wi) in taps:
                acc = acc + jnp.dot(x_ref[0, bi, off:off + ROWS, :],
                                    w_ref[wi], preferred_element_type=F32)
            acc = acc + b_ref[...]
            if act is not None:
                acc = act(acc)
            if pad_out:
                acc = jnp.where(msk, acc, 0.0).astype(out_dtype)
                o_ref[0, bo, 0:OFF, :] = jnp.zeros((OFF, cout), out_dtype)
                o_ref[0, bo, OFF:OFF + ROWS, :] = acc
                o_ref[0, bo, OFF + ROWS:R, :] = jnp.zeros((R - OFF - ROWS, cout), out_dtype)
            else:
                o_ref[0, bo] = acc.astype(out_dtype)

    return pl.pallas_call(
        kfn,
        grid=(B,),
        in_specs=[
            pl.BlockSpec((1, nbi, R, cin), lambda b: (b, 0, 0, 0)),
            pl.BlockSpec((T, cin, cout), lambda b: (0, 0, 0)),
            pl.BlockSpec((1, cout), lambda b: (0, 0)),
        ],
        out_specs=pl.BlockSpec((1, nbo, rout, cout), lambda b: (b, 0, 0, 0)),
        out_shape=jax.ShapeDtypeStruct((B, nbo, rout, cout), out_dtype),
    )(x, w_stack, b_row)


def _enc1(x_blk, w_oihw, b):
    cout = w_oihw.shape[0]
    ws, block_taps = [], []
    for Pr in range(4):
        for Pc in range(4):
            groups = {}
            for kh in range(4):
                t = 2 * Pr + kh - 1
                pir, dr = t % 8, t // 8 + 1
                for kw in range(4):
                    u = 2 * Pc + kw - 1
                    pic, dc = u % 8, u // 8 + 1
                    groups.setdefault((dr, dc), []).append((kh, kw, pir, pic))
            tl = []
            for (dr, dc), lst in groups.items():
                w192 = jnp.zeros((192, cout), F32)
                for kh, kw, pir, pic in lst:
                    c0 = pir * 24 + pic * 3
                    w192 = w192.at[c0:c0 + 3, :].set(
                        jnp.transpose(w_oihw[:, :, kh, kw]))
                tl.append((0, dr * 30 + dc, len(ws)))
                ws.append(w192)
            block_taps.append(tl)
    return _blk_conv(x_blk, jnp.stack(ws), b[None, :], block_taps,
                     jax.nn.relu, cout, out_dtype=x_blk.dtype)


def _enc_s2(h, w_oihw, b, ki, act):
    cout = w_oihw.shape[0]
    n = 1 << ki
    ws, block_taps = [], []
    wcache = {}
    for kh in range(4):
        for kw in range(4):
            wcache[(kh, kw)] = len(ws)
            ws.append(jnp.transpose(w_oihw[:, :, kh, kw]))
    for Pr in range(n // 2):
        for Pc in range(n // 2):
            tl = []
            for kh in range(4):
                t = 2 * Pr + kh - 1
                pr, dr = t % n, t // n + 1
                for kw in range(4):
                    u = 2 * Pc + kw - 1
                    pc, dc = u % n, u // n + 1
                    tl.append((pr * n + pc, dr * 30 + dc, wcache[(kh, kw)]))
            block_taps.append(tl)
    return _blk_conv(h, jnp.stack(ws), b[None, :], block_taps, act, cout,
                     out_dtype=h.dtype)


def _conv_s1(h, w_oihw, b, act, out_dtype=F32):
    cout = w_oihw.shape[0]
    ws, tl = [], []
    for kh in range(3):
        for kw in range(3):
            tl.append((0, kh * 30 + kw, len(ws)))
            ws.append(jnp.transpose(w_oihw[:, :, kh, kw]))
    return _blk_conv(h, jnp.stack(ws), b[None, :], [tl], act, cout,
                     out_dtype=out_dtype)


def _convT(h, w_iokk, b, ki, act):
    cout = w_iokk.shape[1]
    n = 1 << ki
    no = n * 2
    ws, block_taps = [], []
    wcache = {}
    for Pr in range(no):
        for Pc in range(no):
            tl = []
            for kh in range((Pr + 1) % 2, 4, 2):
                s = (Pr - kh + 1) // 2
                pr, dr = s % n, s // n + 1
                for kw in range((Pc + 1) % 2, 4, 2):
                    u = (Pc - kw + 1) // 2
                    pc, dc = u % n, u // n + 1
                    if (kh, kw) not in wcache:
                        wcache[(kh, kw)] = len(ws)
                        ws.append(w_iokk[:, :, kh, kw])
                    tl.append((pr * n + pc, dr * 30 + dc, wcache[(kh, kw)]))
            block_taps.append(tl)
    return _blk_conv(h, jnp.stack(ws), b[None, :], block_taps, act, cout,
                     out_dtype=h.dtype)


def _decT_final(h, w_iokk, b):
    cin = w_iokk.shape[0]
    ws, block_taps = [], []
    for Pr in range(8):
        tl = []
        for kh in range((Pr + 1) % 2, 4, 2):
            s = (Pr - kh + 1) // 2
            pr, dr = s % 4, s // 4 + 1
            groups = {}
            for pic in range(8):
                for kw in range((pic + 1) % 2, 4, 2):
                    u = (pic - kw + 1) // 2
                    pc, dc = u % 4, u // 4 + 1
                    groups.setdefault((pc, dc), []).append((kw, pic))
            for (pc, dc), lst in groups.items():
                w64 = jnp.zeros((cin, 64), w_iokk.dtype)
                for kw, pic in lst:
                    w64 = w64.at[:, pic * 8:pic * 8 + 3].set(w_iokk[:, :, kh, kw])
                tl.append((pr * 4 + pc, dr * 30 + dc, len(ws)))
                ws.append(w64)
        block_taps.append(tl)
    bb = jnp.tile(jnp.pad(b, (0, 5)), 8)[None, :]
    return _blk_conv(h, jnp.stack(ws), bb, block_taps, jax.nn.sigmoid,
                     64, pad_out=False)


def _vq(z2, cb):
    M, D = z2.shape
    K = cb.shape[0]
    CH = 1024
    NC = K // CH
    NVALID = 2 * 28 * 28
    BF16 = jnp.bfloat16
    dn = (((1,), (1,)), ((), ()))

    def kfn(z_ref, cb_ref, zq_ref, vq_ref, pp_ref):
        zv = z_ref[...]
        kk = jax.lax.broadcasted_iota(jnp.int32, (M, 1), 0)
        r = kk % R
        valid = (r >= OFF) & (r < OFF + ROWS) & (((r - OFF) % 30) < 28)
        ones_row = jnp.ones((1, D), F32)

        def body1(k, carry):
            best, bidx = carry
            cbc = cb_ref[pl.ds(k * CH, CH), :]
            cn = jax.lax.dot_general(ones_row, cbc * cbc, dn,
                                     preferred_element_type=F32)
            s = cn - 2.0 * jax.lax.dot_general(zv, cbc, dn,
                                               preferred_element_type=F32)
            m = jnp.min(s, axis=1, keepdims=True)
            am = jnp.argmin(s, axis=1).astype(jnp.int32)[:, None] + k * CH
            upd = m < best
            return (jnp.where(upd, m, best), jnp.where(upd, am, bidx))

        best, bidx = jax.lax.fori_loop(
            0, NC, body1,
            (jnp.full((M, 1), jnp.inf, F32), jnp.zeros((M, 1), jnp.int32)),
            unroll=2)
        zsq = jnp.sum(zv * zv)
        commit = (jnp.sum(jnp.where(valid, best, 0.0)) + zsq) / (NVALID * D)
        vq_ref[...] = jnp.full((1, 1), 0.25 * commit, F32)

        def body2(k, carry):
            zq, ent = carry
            iot = jax.lax.broadcasted_iota(jnp.int32, (1, CH), 1) + k * CH
            oh = ((bidx == iot) & valid).astype(F32)
            zq = zq + jnp.dot(oh, cb_ref[pl.ds(k * CH, CH), :],
                              preferred_element_type=F32)
            p = jnp.sum(oh, axis=0, keepdims=True) * (1.0 / NVALID)
            return (zq, ent + jnp.sum(p * jnp.log(p + 1e-10)))

        zq, ent = jax.lax.fori_loop(
            0, NC, body2, (jnp.zeros((M, D), F32), jnp.zeros((), F32)),
            unroll=2)
        zq_ref[...] = zq
        pp_ref[...] = jnp.full((1, 1), jnp.exp(-ent), F32)

    return pl.pallas_call(
        kfn,
        out_shape=(jax.ShapeDtypeStruct((M, D), F32),
                   jax.ShapeDtypeStruct((1, 1), F32),
                   jax.ShapeDtypeStruct((1, 1), F32)),
    )(z2, cb)


def kernel(x, enc_w1, enc_b1, enc_w2, enc_b2, enc_w3, enc_b3, enc_w4, enc_b4,
           dec_w1, dec_b1, dec_w2, dec_b2, dec_w3, dec_b3, dec_w4, dec_b4,
           codebook):
    relu = jax.nn.relu
    B = x.shape[0]

    xb = x.reshape(B, 3, 28, 8, 28, 8)
    xb = xb.transpose(0, 2, 4, 3, 5, 1).reshape(B, 28, 28, 192)
    xb = jnp.pad(xb, ((0, 0), (1, 2), (1, 1), (0, 0)))
    xb = xb.reshape(B, 1, R, 192)

    h = _enc1(xb, enc_w1, enc_b1)
    h = _enc_s2(h, enc_w2, enc_b2, 2, relu)
    h = _enc_s2(h, enc_w3, enc_b3, 1, relu)
    z_e = _conv_s1(h, enc_w4, enc_b4, None)

    z2 = z_e.reshape(B * R, 64)
    zq2, vq_l, perp = _vq(z2, codebook)
    z_q = zq2.astype(jnp.bfloat16).reshape(B, 1, R, 64)

    h = _conv_s1(z_q, dec_w1, dec_b1, relu,
                 out_dtype=jnp.bfloat16)
    h = _convT(h, dec_w2, dec_b2, 0, relu)
    h = _convT(h, dec_w3, dec_b3, 1, relu)
    o = _decT_final(h, dec_w4, dec_b4)

    o = o.reshape(B, 8, 28, 30, 8, 8)[:, :, :, :28, :, :3]
    x_rec = o.transpose(0, 5, 2, 1, 3, 4).reshape(B, 3, 224, 224)

    return (x_rec, vq_l.reshape(()), perp.reshape(()))


# speedup vs baseline: 1.0556x; 1.0556x over previous
import jax
import jax.numpy as jnp
from jax.experimental import pallas as pl

F32 = jnp.float32
R = 930
ROWS = 840
OFF = 31


def _blk_conv(x, w_stack, b_row, block_taps, act, cout, pad_out=True,
              out_dtype=F32):
    B, nbi, _, cin = x.shape
    w_stack = w_stack.astype(x.dtype)
    nbo = len(block_taps)
    T = w_stack.shape[0]
    rout = R if pad_out else ROWS

    def kfn(x_ref, w_ref, b_ref, o_ref):
        kk = jax.lax.broadcasted_iota(jnp.int32, (ROWS, 1), 0)
        msk = (kk % 30) < 28
        for bo, taps in enumerate(block_taps):
            acc = jnp.zeros((ROWS, cout), F32)
            for (bi, off, wi) in taps:
                acc = acc + jnp.dot(x_ref[0, bi, off:off + ROWS, :],
                                    w_ref[wi], preferred_element_type=F32)
            acc = acc + b_ref[...]
            if act is not None:
                acc = act(acc)
            if pad_out:
                acc = jnp.where(msk, acc, 0.0).astype(out_dtype)
                o_ref[0, bo, 0:OFF, :] = jnp.zeros((OFF, cout), out_dtype)
                o_ref[0, bo, OFF:OFF + ROWS, :] = acc
                o_ref[0, bo, OFF + ROWS:R, :] = jnp.zeros((R - OFF - ROWS, cout), out_dtype)
            else:
                o_ref[0, bo] = acc.astype(out_dtype)

    return pl.pallas_call(
        kfn,
        grid=(B,),
        in_specs=[
            pl.BlockSpec((1, nbi, R, cin), lambda b: (b, 0, 0, 0)),
            pl.BlockSpec((T, cin, cout), lambda b: (0, 0, 0)),
            pl.BlockSpec((1, cout), lambda b: (0, 0)),
        ],
        out_specs=pl.BlockSpec((1, nbo, rout, cout), lambda b: (b, 0, 0, 0)),
        out_shape=jax.ShapeDtypeStruct((B, nbo, rout, cout), out_dtype),
    )(x, w_stack, b_row)


def _enc1(x_blk, w_oihw, b):
    cout = w_oihw.shape[0]
    ws, block_taps = [], []
    for Pr in range(4):
        for Pc in range(4):
            tl = []
            for kh in range(4):
                t = 2 * Pr + kh - 1
                pr, dr = t % 8, t // 8 + 1
                groups = {}
                for kw in range(4):
                    u = 2 * Pc + kw - 1
                    pic, dc = u % 8, u // 8 + 1
                    groups.setdefault(dc, []).append((kw, pic))
                for dc, lst in groups.items():
                    w24 = [jnp.zeros((3, cout), F32)] * 8
                    for kw, pic in lst:
                        w24[pic] = jnp.transpose(w_oihw[:, :, kh, kw])
                    tl.append((pr, dr * 30 + dc, len(ws)))
                    ws.append(jnp.concatenate(w24, axis=0))
            block_taps.append(tl)
    return _blk_conv(x_blk, jnp.stack(ws), b[None, :], block_taps,
                     jax.nn.relu, cout, out_dtype=x_blk.dtype)


def _enc_s2(h, w_oihw, b, ki, act):
    cout = w_oihw.shape[0]
    n = 1 << ki
    ws, block_taps = [], []
    wcache = {}
    for kh in range(4):
        for kw in range(4):
            wcache[(kh, kw)] = len(ws)
            ws.append(jnp.transpose(w_oihw[:, :, kh, kw]))
    for Pr in range(n // 2):
        for Pc in range(n // 2):
            tl = []
            for kh in range(4):
                t = 2 * Pr + kh - 1
                pr, dr = t % n, t // n + 1
                for kw in range(4):
                    u = 2 * Pc + kw - 1
                    pc, dc = u % n, u // n + 1
                    tl.append((pr * n + pc, dr * 30 + dc, wcache[(kh, kw)]))
            block_taps.append(tl)
    return _blk_conv(h, jnp.stack(ws), b[None, :], block_taps, act, cout,
                     out_dtype=h.dtype)


def _conv_s1(h, w_oihw, b, act, out_dtype=F32):
    cout = w_oihw.shape[0]
    ws, tl = [], []
    for kh in range(3):
        for kw in range(3):
            tl.append((0, kh * 30 + kw, len(ws)))
            ws.append(jnp.transpose(w_oihw[:, :, kh, kw]))
    return _blk_conv(h, jnp.stack(ws), b[None, :], [tl], act, cout,
                     out_dtype=out_dtype)


def _convT(h, w_iokk, b, ki, act):
    cout = w_iokk.shape[1]
    n = 1 << ki
    no = n * 2
    ws, block_taps = [], []
    wcache = {}
    for Pr in range(no):
        for Pc in range(no):
            tl = []
            for kh in range((Pr + 1) % 2, 4, 2):
                s = (Pr - kh + 1) // 2
                pr, dr = s % n, s // n + 1
                for kw in range((Pc + 1) % 2, 4, 2):
                    u = (Pc - kw + 1) // 2
                    pc, dc = u % n, u // n + 1
                    if (kh, kw) not in wcache:
                        wcache[(kh, kw)] = len(ws)
                        ws.append(w_iokk[:, :, kh, kw])
                    tl.append((pr * n + pc, dr * 30 + dc, wcache[(kh, kw)]))
            block_taps.append(tl)
    return _blk_conv(h, jnp.stack(ws), b[None, :], block_taps, act, cout,
                     out_dtype=h.dtype)


def _decT_final(h, w_iokk, b):
    cin = w_iokk.shape[0]
    ws, block_taps = [], []
    for Pr in range(8):
        tl = []
        for kh in range((Pr + 1) % 2, 4, 2):
            s = (Pr - kh + 1) // 2
            pr, dr = s % 4, s // 4 + 1
            groups = {}
            for pic in range(8):
                for kw in range((pic + 1) % 2, 4, 2):
                    u = (pic - kw + 1) // 2
                    pc, dc = u % 4, u // 4 + 1
                    groups.setdefault((pc, dc), []).append((kw, pic))
            for (pc, dc), lst in groups.items():
                w64 = jnp.zeros((cin, 64), w_iokk.dtype)
                for kw, pic in lst:
                    w64 = w64.at[:, pic * 8:pic * 8 + 3].set(w_iokk[:, :, kh, kw])
                tl.append((pr * 4 + pc, dr * 30 + dc, len(ws)))
                ws.append(w64)
        block_taps.append(tl)
    bb = jnp.tile(jnp.pad(b, (0, 5)), 8)[None, :]
    return _blk_conv(h, jnp.stack(ws), bb, block_taps, jax.nn.sigmoid,
                     64, pad_out=False)


def _vq(z2, cb):
    M, D = z2.shape
    K = cb.shape[0]
    CH = 1024
    NC = K // CH
    NVALID = 2 * 28 * 28
    BF16 = jnp.bfloat16
    dn = (((1,), (1,)), ((), ()))

    def kfn(z_ref, cb_ref, zq_ref, vq_ref, pp_ref):
        zv = z_ref[...]
        kk = jax.lax.broadcasted_iota(jnp.int32, (M, 1), 0)
        r = kk % R
        valid = (r >= OFF) & (r < OFF + ROWS) & (((r - OFF) % 30) < 28)
        ones_row = jnp.ones((1, D), F32)

        def body1(k, carry):
            best, bidx = carry
            cbc = cb_ref[pl.ds(k * CH, CH), :]
            cn = jax.lax.dot_general(ones_row, cbc * cbc, dn,
                                     preferred_element_type=F32)
            s = cn - 2.0 * jax.lax.dot_general(zv, cbc, dn,
                                               preferred_element_type=F32)
            m = jnp.min(s, axis=1, keepdims=True)
            am = jnp.argmin(s, axis=1).astype(jnp.int32)[:, None] + k * CH
            upd = m < best
            return (jnp.where(upd, m, best), jnp.where(upd, am, bidx))

        best, bidx = jax.lax.fori_loop(
            0, NC, body1,
            (jnp.full((M, 1), jnp.inf, F32), jnp.zeros((M, 1), jnp.int32)),
            unroll=2)
        zsq = jnp.sum(zv * zv)
        commit = (jnp.sum(jnp.where(valid, best, 0.0)) + zsq) / (NVALID * D)
        vq_ref[...] = jnp.full((1, 1), 0.25 * commit, F32)

        def body2(k, carry):
            zq, ent = carry
            iot = jax.lax.broadcasted_iota(jnp.int32, (1, CH), 1) + k * CH
            oh = ((bidx == iot) & valid).astype(F32)
            zq = zq + jnp.dot(oh, cb_ref[pl.ds(k * CH, CH), :],
                              preferred_element_type=F32)
            p = jnp.sum(oh, axis=0, keepdims=True) * (1.0 / NVALID)
            return (zq, ent + jnp.sum(p * jnp.log(p + 1e-10)))

        zq, ent = jax.lax.fori_loop(
            0, NC, body2, (jnp.zeros((M, D), F32), jnp.zeros((), F32)),
            unroll=2)
        zq_ref[...] = zq
        pp_ref[...] = jnp.full((1, 1), jnp.exp(-ent), F32)

    return pl.pallas_call(
        kfn,
        out_shape=(jax.ShapeDtypeStruct((M, D), F32),
                   jax.ShapeDtypeStruct((1, 1), F32),
                   jax.ShapeDtypeStruct((1, 1), F32)),
    )(z2, cb)


def kernel(x, enc_w1, enc_b1, enc_w2, enc_b2, enc_w3, enc_b3, enc_w4, enc_b4,
           dec_w1, dec_b1, dec_w2, dec_b2, dec_w3, dec_b3, dec_w4, dec_b4,
           codebook):
    relu = jax.nn.relu
    B = x.shape[0]

    xb = x.reshape(B, 3, 28, 8, 28, 8)
    xb = xb.transpose(0, 3, 2, 4, 5, 1).reshape(B, 8, 28, 28, 24)
    xb = jnp.pad(xb, ((0, 0), (0, 0), (1, 2), (1, 1), (0, 0)))
    xb = xb.reshape(B, 8, R, 24)

    h = _enc1(xb, enc_w1, enc_b1)
    h = _enc_s2(h, enc_w2, enc_b2, 2, relu)
    h = _enc_s2(h, enc_w3, enc_b3, 1, relu)
    z_e = _conv_s1(h, enc_w4, enc_b4, None)

    z2 = z_e.reshape(B * R, 64)
    zq2, vq_l, perp = _vq(z2, codebook)
    z_q = zq2.astype(jnp.bfloat16).reshape(B, 1, R, 64)

    h = _conv_s1(z_q, dec_w1, dec_b1, relu,
                 out_dtype=jnp.bfloat16)
    h = _convT(h, dec_w2, dec_b2, 0, relu)
    h = _convT(h, dec_w3, dec_b3, 1, relu)
    o = _decT_final(h, dec_w4, dec_b4)

    o = o.reshape(B, 8, 28, 30, 8, 8)[:, :, :, :28, :, :3]
    x_rec = o.transpose(0, 5, 2, 1, 3, 4).reshape(B, 3, 224, 224)

    return (x_rec, vq_l.reshape(()), perp.reshape(()))
